# baseline (speedup 1.0000x reference)
import functools

import jax
import jax.numpy as jnp
from jax import lax
from jax.experimental import pallas as pl
from jax.experimental.pallas import tpu as pltpu
from jax.experimental.pallas import tpu_sc as plsc

DIM_ = 4096
L_ = 16
NW_ = 32
CHUNK_ = 8


def _sc_body(idx_hbm, feat_hbm, gamma_hbm, beta_hbm, out_hbm,
             idx_v, g_v, b_v, buf0, buf1, sem0, sem1):
    B = feat_hbm.shape[0]
    rows_per_tile = B // NW_
    n_chunks = rows_per_tile // CHUNK_
    wid = lax.axis_index("s") * 2 + lax.axis_index("c")
    base = wid * rows_per_tile

    pltpu.sync_copy(idx_hbm, idx_v)
    pltpu.async_copy(gamma_hbm.at[idx_v], g_v, sem0).wait()
    pltpu.async_copy(beta_hbm.at[idx_v], b_v, sem0).wait()

    bufs = (buf0, buf1)
    sems = (sem0, sem1)

    def compute(buf):
        def col_step(k, _):
            sl = pl.ds(k * L_, L_)
            for r in range(CHUNK_):
                buf[r, sl] = buf[r, sl] * g_v[0, sl] + b_v[0, sl]
            return 0
        lax.fori_loop(0, DIM_ // L_, col_step, 0)

    pltpu.async_copy(feat_hbm.at[pl.ds(base, CHUNK_)], buf0, sem0)
    pltpu.async_copy(feat_hbm.at[pl.ds(base + CHUNK_, CHUNK_)], buf1, sem1)

    def chunk_step(c, _):
        for p in range(2):
            cc = 2 * c + p
            buf, sem = bufs[p], sems[p]
            pltpu.make_async_copy(feat_hbm.at[pl.ds(base, CHUNK_)], buf, sem).wait()
            compute(buf)
            r0 = base + cc * CHUNK_
            pltpu.async_copy(buf, out_hbm.at[pl.ds(r0, CHUNK_)], sem)
            pltpu.make_async_copy(buf, out_hbm.at[pl.ds(r0, CHUNK_)], sem).wait()
            nxt = cc + 2
            @pl.when(nxt < n_chunks)
            def _():
                pltpu.async_copy(
                    feat_hbm.at[pl.ds(base + nxt * CHUNK_, CHUNK_)], buf, sem)
        return 0

    lax.fori_loop(0, n_chunks // 2, chunk_step, 0)


def kernel(feat, modality_id, gamma, beta):
    B, D = feat.shape
    idx = jnp.asarray(modality_id, jnp.int32).reshape(1)
    mesh = plsc.VectorSubcoreMesh(core_axis_name="c", subcore_axis_name="s")
    f = functools.partial(
        pl.kernel,
        out_type=jax.ShapeDtypeStruct((B, D), jnp.float32),
        mesh=mesh,
        scratch_types=[
            pltpu.VMEM((1,), jnp.int32),
            pltpu.VMEM((1, D), jnp.float32),
            pltpu.VMEM((1, D), jnp.float32),
            pltpu.VMEM((CHUNK_, D), jnp.float32),
            pltpu.VMEM((CHUNK_, D), jnp.float32),
            pltpu.SemaphoreType.DMA,
            pltpu.SemaphoreType.DMA,
        ],
    )(_sc_body)
    return f(idx, feat, gamma, beta)


# speedup vs baseline: 4.7739x; 4.7739x over previous
import functools

import jax
import jax.numpy as jnp
from jax import lax
from jax.experimental import pallas as pl
from jax.experimental.pallas import tpu as pltpu
from jax.experimental.pallas import tpu_sc as plsc

DIM_ = 4096
BM_ = 512


def _scs_gather_body(idx_hbm, gamma_hbm, beta_hbm, g_out, b_out, idx_s):
    cid = lax.axis_index("c")

    @pl.when(cid == 0)
    def _():
        pltpu.sync_copy(idx_hbm, idx_s)
        i = idx_s[0]
        pltpu.sync_copy(gamma_hbm.at[pl.ds(i, 1)], g_out)
        pltpu.sync_copy(beta_hbm.at[pl.ds(i, 1)], b_out)


def _sc_gather(idx, gamma, beta):
    D = gamma.shape[1]
    mesh = plsc.ScalarSubcoreMesh(axis_name="c", num_cores=2)
    f = functools.partial(
        pl.kernel,
        out_type=[
            jax.ShapeDtypeStruct((1, D), jnp.float32),
            jax.ShapeDtypeStruct((1, D), jnp.float32),
        ],
        mesh=mesh,
        scratch_types=[
            pltpu.SMEM((1,), jnp.int32),
        ],
    )(_scs_gather_body)
    return f(idx, gamma, beta)


def _affine_body(feat_ref, g_ref, b_ref, out_ref):
    out_ref[...] = feat_ref[...] * g_ref[...] + b_ref[...]


def kernel(feat, modality_id, gamma, beta):
    B, D = feat.shape
    idx = jnp.asarray(modality_id, jnp.int32).reshape(1)
    g_row, b_row = _sc_gather(idx, gamma, beta)
    grid = (B // BM_,)
    return pl.pallas_call(
        _affine_body,
        grid=grid,
        in_specs=[
            pl.BlockSpec((BM_, D), lambda i: (i, 0)),
            pl.BlockSpec((1, D), lambda i: (0, 0)),
            pl.BlockSpec((1, D), lambda i: (0, 0)),
        ],
        out_specs=pl.BlockSpec((BM_, D), lambda i: (i, 0)),
        out_shape=jax.ShapeDtypeStruct((B, D), feat.dtype),
        compiler_params=pltpu.CompilerParams(
            dimension_semantics=("arbitrary",),
        ),
    )(feat, g_row, b_row)
